# baseline (speedup 1.0000x reference)
import functools

import jax
import jax.numpy as jnp
from jax.experimental import pallas as pl
from jax.experimental.pallas import tpu as pltpu
from jax.experimental.pallas import tpu_sc as plsc

_BATCH = 64
_TOPK = 512
_DIM = 1024
_HID = 2048
_MEM = 128
_BB1 = 8
_BB2 = 4
_NC = 2
_NW = 32
_BPW = 256
_CH = 64
_NCH = 4
_NIDX = _BATCH * _MEM


def _topk_kernel(w_ref, idx_ref, sw_ref):
    w = jnp.maximum(w_ref[...], 0.0)
    s = jnp.maximum(jnp.sum(w, axis=2, keepdims=True), 1e-6)
    wn = w / s

    wn_col = jnp.transpose(wn, (0, 2, 1))
    wi = jnp.broadcast_to(wn_col, (_BB1, _TOPK, _TOPK))
    wj = jnp.broadcast_to(wn, (_BB1, _TOPK, _TOPK))
    ii = jax.lax.broadcasted_iota(jnp.int32, (_BB1, _TOPK, _TOPK), 1)
    jj = jax.lax.broadcasted_iota(jnp.int32, (_BB1, _TOPK, _TOPK), 2)
    beats = (wi > wj) | ((wi == wj) & (ii < jj))
    rank = jnp.sum(beats.astype(jnp.int32), axis=1, keepdims=True)

    t_iota = jax.lax.broadcasted_iota(jnp.int32, (_BB1, _MEM, _TOPK), 1)
    eq = jnp.broadcast_to(rank, (_BB1, _MEM, _TOPK)) == t_iota

    sw_ref[...] = jnp.sum(jnp.where(eq, jnp.broadcast_to(wn, (_BB1, _MEM, _TOPK)), 0.0),
                          axis=2)

    j_i = jax.lax.broadcasted_iota(jnp.int32, (_BB1, _MEM, _TOPK), 2)
    idx_local = jnp.sum(jnp.where(eq, j_i, 0), axis=2)
    base = (pl.program_id(0) * _BB1
            + jax.lax.broadcasted_iota(jnp.int32, (_BB1, _MEM), 0)) * _TOPK
    idx_ref[...] = idx_local + base


def _proj_kernel(sel_ref, sw_ref, wt_ref, b_ref, g_ref, bt_ref, out_ref):
    sel = jnp.clip(sel_ref[...], -5.0, 5.0)
    tokens = jax.lax.dot_general(
        sel, wt_ref[...], (((1,), (1,)), ((), ())),
        preferred_element_type=jnp.float32,
        precision=jax.lax.Precision.DEFAULT)
    tokens = (tokens + b_ref[...]) * sw_ref[...]
    tokens = jnp.clip(tokens, -5.0, 5.0)
    mean = jnp.mean(tokens, axis=-1, keepdims=True)
    cent = tokens - mean
    var = jnp.mean(cent * cent, axis=-1, keepdims=True)
    out = cent * jax.lax.rsqrt(var + 1e-5) * g_ref[...] + bt_ref[...]
    out_ref[...] = out.reshape(_BB2, _MEM, _HID)


def _sc_gather(e_flat, idx_flat):
    mesh = plsc.VectorSubcoreMesh(core_axis_name="c", subcore_axis_name="s")

    @functools.partial(
        pl.kernel, mesh=mesh,
        out_type=jax.ShapeDtypeStruct((_NIDX, _DIM), jnp.float32),
        scratch_types=[
            pltpu.VMEM((_CH,), jnp.int32),
            pltpu.VMEM((_CH, _DIM), jnp.float32),
            pltpu.SemaphoreType.DMA,
        ],
    )
    def k(table_hbm, idx_hbm, out_hbm, idx_v, rows_v, sem):
        wid = jax.lax.axis_index("s") * _NC + jax.lax.axis_index("c")
        base = wid * _BPW

        @pl.loop(0, _NCH)
        def _(c):
            off = base + c * _CH
            pltpu.sync_copy(idx_hbm.at[pl.ds(off, _CH)], idx_v)
            pltpu.async_copy(table_hbm.at[idx_v], rows_v, sem).wait()
            pltpu.sync_copy(rows_v, out_hbm.at[pl.ds(off, _CH)])

    return k(e_flat, idx_flat)


@jax.jit
def kernel(image_embeds, weights, W, b, gamma, beta):
    b2 = b.reshape(1, _HID)
    g2 = gamma.reshape(1, _HID)
    bt2 = beta.reshape(1, _HID)

    idx, sw = pl.pallas_call(
        _topk_kernel,
        grid=(_BATCH // _BB1,),
        in_specs=[pl.BlockSpec((_BB1, 1, _TOPK), lambda i: (i, 0, 0))],
        out_specs=[
            pl.BlockSpec((_BB1, _MEM), lambda i: (i, 0)),
            pl.BlockSpec((_BB1, _MEM), lambda i: (i, 0)),
        ],
        out_shape=[
            jax.ShapeDtypeStruct((_BATCH, _MEM), jnp.int32),
            jax.ShapeDtypeStruct((_BATCH, _MEM), jnp.float32),
        ],
        compiler_params=pltpu.CompilerParams(
            dimension_semantics=("arbitrary",),
        ),
    )(weights.reshape(_BATCH, 1, _TOPK))

    sel = _sc_gather(image_embeds.reshape(_BATCH * _TOPK, _DIM),
                     idx.reshape(_NIDX))

    return pl.pallas_call(
        _proj_kernel,
        grid=(_BATCH // _BB2,),
        in_specs=[
            pl.BlockSpec((_BB2 * _MEM, _DIM), lambda i: (i, 0)),
            pl.BlockSpec((_BB2 * _MEM, 1), lambda i: (i, 0)),
            pl.BlockSpec((_HID, _DIM), lambda i: (0, 0)),
            pl.BlockSpec((1, _HID), lambda i: (0, 0)),
            pl.BlockSpec((1, _HID), lambda i: (0, 0)),
            pl.BlockSpec((1, _HID), lambda i: (0, 0)),
        ],
        out_specs=pl.BlockSpec((_BB2, _MEM, _HID), lambda i: (i, 0, 0)),
        out_shape=jax.ShapeDtypeStruct((_BATCH, _MEM, _HID), jnp.float32),
        compiler_params=pltpu.CompilerParams(
            dimension_semantics=("arbitrary",),
        ),
    )(sel, sw.reshape(_NIDX, 1), W, b2, g2, bt2)
